# Initial kernel scaffold; baseline (speedup 1.0000x reference)
import functools

import jax
import jax.numpy as jnp
from jax import lax
from jax.experimental import pallas as pl
from jax.experimental.pallas import tpu as pltpu
from jax.experimental.pallas import tpu_sc as plsc

N_ROWS = 320000
D = 128
NSEG = 10000
NW = 32
SPW = 313
SEG_PAD = NW * SPW
CHUNK = 256
NB = 48


def _sc_body(bounds_hbm, seq_hbm, ids_hbm, out_hbm,
             bounds_v, ids_v, in_v, acc_v, cnt_v):
    wid = lax.axis_index("s") * 2 + lax.axis_index("c")

    pltpu.sync_copy(bounds_hbm, bounds_v)

    zeros = jnp.zeros((16,), jnp.float32)
    ones = jnp.ones((16,), jnp.float32)

    def zero_acc(q, _):
        acc_v[pl.ds(q * 16, 16)] = zeros
        return 0
    lax.fori_loop(0, SPW * D // 16, zero_acc, 0)

    def zero_cnt(q, _):
        cnt_v[pl.ds(q * 16, 16)] = zeros
        return 0
    lax.fori_loop(0, SPW, zero_cnt, 0)

    seg_base = wid * SPW
    r0 = bounds_v[wid]
    r1 = bounds_v[wid + 1]
    a0 = (r0 >> 3) << 3
    nchunks = (r1 - a0 + (CHUNK - 1)) >> 8

    def chunk_body(j, _):
        base = a0 + j * CHUNK
        start = jnp.minimum(base, N_ROWS - CHUNK)
        pltpu.sync_copy(ids_hbm.at[pl.ds(start, CHUNK)], ids_v)
        pltpu.sync_copy(seq_hbm.at[pl.ds(start * D, CHUNK * D)], in_v)
        lo = jnp.maximum(r0, base) - start
        hi = jnp.minimum(r1, start + CHUNK) - start

        def row_body(i, _):
            sid = ids_v[i]
            local = sid - seg_base
            off = local * D
            for k in range(D // 16):
                acc_v[pl.ds(off + k * 16, 16)] += in_v[pl.ds(i * D + k * 16, 16)]
            cnt_v[pl.ds(local * 16, 16)] += ones
            return 0
        lax.fori_loop(lo, hi, row_body, 0)
        return 0
    lax.fori_loop(0, nchunks, chunk_body, 0)

    def div_body(s, _):
        cv = cnt_v[pl.ds(s * 16, 16)]
        scale = 1.0 / (cv + 1.0)
        for k in range(D // 16):
            acc_v[pl.ds(s * D + k * 16, 16)] *= scale
        return 0
    lax.fori_loop(0, SPW, div_body, 0)

    pltpu.sync_copy(acc_v, out_hbm.at[pl.ds(seg_base * D, SPW * D)])


@jax.jit
def _sc_call(bounds, seqf, ids):
    mesh = plsc.VectorSubcoreMesh(core_axis_name="c", subcore_axis_name="s")
    return pl.kernel(
        _sc_body,
        mesh=mesh,
        out_type=jax.ShapeDtypeStruct((SEG_PAD * D,), jnp.float32),
        scratch_types=[
            pltpu.VMEM((NB,), jnp.int32),
            pltpu.VMEM((CHUNK,), jnp.int32),
            pltpu.VMEM((CHUNK * D,), jnp.float32),
            pltpu.VMEM((SPW * D,), jnp.float32),
            pltpu.VMEM((SPW * 16,), jnp.float32),
        ],
    )(bounds, seqf, ids)


def kernel(seq, sub_match):
    ids = sub_match.astype(jnp.int32)
    marks = jnp.arange(NW + 1, dtype=jnp.int32) * SPW
    bounds = jnp.searchsorted(ids, marks).astype(jnp.int32)
    bounds = jnp.pad(bounds, (0, NB - (NW + 1)))
    out = _sc_call(bounds, seq.reshape(-1), ids)
    return out.reshape(SEG_PAD, D)[:NSEG]


# speedup vs baseline: 1.8598x; 1.8598x over previous
import functools

import jax
import jax.numpy as jnp
from jax import lax
from jax.experimental import pallas as pl
from jax.experimental.pallas import tpu as pltpu
from jax.experimental.pallas import tpu_sc as plsc

N_ROWS = 320000
D = 128
NSEG = 10000
NW = 32
SPW = 313
SEG_PAD = NW * SPW
CHUNK = 256
NB = 48


def _sc_body(bounds_hbm, seq_hbm, ids_hbm, out_hbm,
             bounds_v, ids_v, in_v, acc_v, cnt_v):
    wid = lax.axis_index("s") * 2 + lax.axis_index("c")

    pltpu.sync_copy(bounds_hbm, bounds_v)

    zeros = jnp.zeros((16,), jnp.float32)
    ones = jnp.ones((16,), jnp.float32)

    def zero_acc(q, _):
        acc_v[pl.ds(q * 16, 16)] = zeros
        return 0
    lax.fori_loop(0, SPW * D // 16, zero_acc, 0)

    def zero_cnt(q, _):
        cnt_v[pl.ds(q * 16, 16)] = zeros
        return 0
    lax.fori_loop(0, SPW, zero_cnt, 0)

    seg_base = wid * SPW
    bv = bounds_v[pl.ds(wid, 16)]
    r0 = bv[0]
    r1 = bv[1]
    a0 = (r0 >> 3) << 3
    nchunks = (r1 - a0 + (CHUNK - 1)) >> 8

    def chunk_body(j, _):
        base = a0 + j * CHUNK
        start = pl.multiple_of(jnp.minimum(base, N_ROWS - CHUNK), 8)
        pltpu.sync_copy(ids_hbm.at[pl.ds(start, CHUNK)], ids_v.at[pl.ds(0, CHUNK)])
        pltpu.sync_copy(seq_hbm.at[pl.ds(start * D, CHUNK * D)], in_v)
        lo = jnp.maximum(r0, base) - start
        hi = jnp.minimum(r1, start + CHUNK) - start

        def row_body(i, _):
            sid = ids_v[pl.ds(i, 16)][0]
            local = sid - seg_base
            off = local * D
            for k in range(D // 16):
                acc_v[pl.ds(off + k * 16, 16)] += in_v[pl.ds(i * D + k * 16, 16)]
            cnt_v[pl.ds(local * 16, 16)] += ones
            return 0
        lax.fori_loop(lo, hi, row_body, 0)
        return 0
    lax.fori_loop(0, nchunks, chunk_body, 0)

    def div_body(s, _):
        cv = cnt_v[pl.ds(s * 16, 16)]
        scale = 1.0 / (cv + 1.0)
        for k in range(D // 16):
            acc_v[pl.ds(s * D + k * 16, 16)] *= scale
        return 0
    lax.fori_loop(0, SPW, div_body, 0)

    pltpu.sync_copy(acc_v, out_hbm.at[pl.ds(seg_base * D, SPW * D)])


@jax.jit
def _sc_call(bounds, seqf, ids):
    mesh = plsc.VectorSubcoreMesh(core_axis_name="c", subcore_axis_name="s")
    return pl.kernel(
        _sc_body,
        mesh=mesh,
        out_type=jax.ShapeDtypeStruct((SEG_PAD * D,), jnp.float32),
        scratch_types=[
            pltpu.VMEM((NB,), jnp.int32),
            pltpu.VMEM((CHUNK + 16,), jnp.int32),
            pltpu.VMEM((CHUNK * D,), jnp.float32),
            pltpu.VMEM((SPW * D,), jnp.float32),
            pltpu.VMEM((SPW * 16,), jnp.float32),
        ],
    )(bounds, seqf, ids)


def kernel(seq, sub_match):
    ids = sub_match.astype(jnp.int32)
    marks = jnp.arange(NW + 1, dtype=jnp.int32) * SPW
    bounds = jnp.searchsorted(ids, marks).astype(jnp.int32)
    bounds = jnp.pad(bounds, (0, NB - (NW + 1)))
    out = _sc_call(bounds, seq.reshape(-1), ids)
    return out.reshape(SEG_PAD, D)[:NSEG]


# speedup vs baseline: 3.6588x; 1.9673x over previous
import jax
import jax.numpy as jnp
from jax import lax
from jax.experimental import pallas as pl
from jax.experimental.pallas import tpu as pltpu
from jax.experimental.pallas import tpu_sc as plsc

N_ROWS = 320000
D = 128
NSEG = 10000
NW = 32
SPW = 313
SEG_PAD = NW * SPW
CHUNK = 256
CSHIFT = 8
NB = 48


def _sc_body(bounds_hbm, seq_hbm, ids_hbm, out_hbm,
             bounds_v, ids0, ids1, in0, in1, acc_v, cnt_v, sem0, sem1):
    wid = lax.axis_index("s") * 2 + lax.axis_index("c")

    pltpu.sync_copy(bounds_hbm, bounds_v)

    zeros = jnp.zeros((16,), jnp.float32)
    ones = jnp.ones((16,), jnp.float32)
    iota = lax.iota(jnp.int32, 16)

    def zero_body(s, _):
        for k in range(D // 16):
            acc_v[pl.ds(s * D + k * 16, 16)] = zeros
        cnt_v[pl.ds(s * 16, 16)] = zeros
        return 0
    lax.fori_loop(0, SPW, zero_body, 0)

    seg_base = wid * SPW
    bv = bounds_v[pl.ds(wid, 16)]
    r0 = bv[0]
    r1 = bv[1]
    a0 = (r0 >> 3) << 3
    nchunks = (r1 - a0 + (CHUNK - 1)) >> CSHIFT

    ids_bufs = (ids0, ids1)
    in_bufs = (in0, in1)
    sems = (sem0, sem1)

    def chunk_start(j):
        return pl.multiple_of(jnp.minimum(a0 + j * CHUNK, N_ROWS - CHUNK), 8)

    def start_dmas(j, b):
        st = chunk_start(j)
        pltpu.async_copy(ids_hbm.at[pl.ds(st, CHUNK)], ids_bufs[b], sems[b])
        pltpu.async_copy(seq_hbm.at[pl.ds(st * D, CHUNK * D)], in_bufs[b], sems[b])

    def wait_dmas(b):
        pltpu.make_async_copy(ids_hbm.at[pl.ds(0, CHUNK)], ids_bufs[b], sems[b]).wait()
        pltpu.make_async_copy(seq_hbm.at[pl.ds(0, CHUNK * D)], in_bufs[b], sems[b]).wait()

    def process(j, b):
        ids_v = ids_bufs[b]
        in_v = in_bufs[b]
        st = chunk_start(j)
        lo = jnp.maximum(r0, a0 + j * CHUNK) - st
        hi = jnp.minimum(r1, st + CHUNK) - st

        def group_body(g, _):
            idv = ids_v[pl.ds(g * 16, 16)]
            rowi = g * 16 + iota
            inr = (rowi >= lo) & (rowi < hi)
            locv = idv - seg_base
            offv = jnp.where(inr, locv * D, SPW * D)
            cofv = jnp.where(inr, locv * 16, SPW * 16)
            for lane in range(16):
                off = offv[lane]
                rbase = (g * 16 + lane) * D
                for k in range(D // 16):
                    plsc.addupdate(acc_v.at[pl.ds(off + k * 16, 16)],
                                   in_v[pl.ds(rbase + k * 16, 16)])
                plsc.addupdate(cnt_v.at[pl.ds(cofv[lane], 16)], ones)
            return 0
        lax.fori_loop(lo >> 4, (hi + 15) >> 4, group_body, 0)

    for b in range(2):
        @pl.when(b < nchunks)
        def _(b=b):
            start_dmas(b, b)

    def pair_body(p, _):
        for b in range(2):
            j = p * 2 + b

            @pl.when(j < nchunks)
            def _(j=j, b=b):
                wait_dmas(b)
                process(j, b)

                @pl.when(j + 2 < nchunks)
                def _(j=j, b=b):
                    start_dmas(j + 2, b)
        return 0
    lax.fori_loop(0, (nchunks + 1) >> 1, pair_body, 0)

    def div_body(s, _):
        cv = cnt_v[pl.ds(s * 16, 16)]
        scale = 1.0 / (cv + 1.0)
        for k in range(D // 16):
            acc_v[pl.ds(s * D + k * 16, 16)] *= scale
        return 0
    lax.fori_loop(0, SPW, div_body, 0)

    pltpu.sync_copy(acc_v.at[pl.ds(0, SPW * D)],
                    out_hbm.at[pl.ds(seg_base * D, SPW * D)])


@jax.jit
def _sc_call(bounds, seqf, ids):
    mesh = plsc.VectorSubcoreMesh(core_axis_name="c", subcore_axis_name="s")
    return pl.kernel(
        _sc_body,
        mesh=mesh,
        out_type=jax.ShapeDtypeStruct((SEG_PAD * D,), jnp.float32),
        scratch_types=[
            pltpu.VMEM((NB,), jnp.int32),
            pltpu.VMEM((CHUNK,), jnp.int32),
            pltpu.VMEM((CHUNK,), jnp.int32),
            pltpu.VMEM((CHUNK * D,), jnp.float32),
            pltpu.VMEM((CHUNK * D,), jnp.float32),
            pltpu.VMEM(((SPW + 1) * D,), jnp.float32),
            pltpu.VMEM(((SPW + 1) * 16,), jnp.float32),
            pltpu.SemaphoreType.DMA,
            pltpu.SemaphoreType.DMA,
        ],
    )(bounds, seqf, ids)


def kernel(seq, sub_match):
    ids = sub_match.astype(jnp.int32)
    marks = jnp.arange(NW + 1, dtype=jnp.int32) * SPW
    bounds = jnp.searchsorted(ids, marks).astype(jnp.int32)
    bounds = jnp.pad(bounds, (0, NB - (NW + 1)))
    out = _sc_call(bounds, seq.reshape(-1), ids)
    return out.reshape(SEG_PAD, D)[:NSEG]


# speedup vs baseline: 3.7199x; 1.0167x over previous
import jax
import jax.numpy as jnp
from jax import lax
from jax.experimental import pallas as pl
from jax.experimental.pallas import tpu as pltpu
from jax.experimental.pallas import tpu_sc as plsc

N_ROWS = 320000
D = 128
NSEG = 10000
NW = 32
SPW = 313
SEG_PAD = NW * SPW
CHUNK = 256
CSHIFT = 8
NB = 48


def _sc_body(bounds_hbm, seq_hbm, ids_hbm, out_hbm,
             bounds_v, ids0, ids1, in0, in1, acc_v, cnt_v, sem0, sem1):
    wid = lax.axis_index("s") * 2 + lax.axis_index("c")

    pltpu.sync_copy(bounds_hbm, bounds_v)

    zeros = jnp.zeros((16,), jnp.float32)
    ones = jnp.ones((16,), jnp.float32)
    iota = lax.iota(jnp.int32, 16)

    def zero_body(s, _):
        for k in range(D // 16):
            acc_v[pl.ds(s * D + k * 16, 16)] = zeros
        cnt_v[pl.ds(s * 16, 16)] = zeros
        return 0
    lax.fori_loop(0, SPW, zero_body, 0)

    seg_base = wid * SPW
    bv = bounds_v[pl.ds(wid, 16)]
    r0 = bv[0]
    r1 = bv[1]
    a0 = (r0 >> 3) << 3
    nchunks = (r1 - a0 + (CHUNK - 1)) >> CSHIFT

    ids_bufs = (ids0, ids1)
    in_bufs = (in0, in1)
    sems = (sem0, sem1)

    def chunk_start(j):
        return pl.multiple_of(jnp.minimum(a0 + j * CHUNK, N_ROWS - CHUNK), 8)

    def start_dmas(j, b):
        st = chunk_start(j)
        pltpu.async_copy(ids_hbm.at[pl.ds(st, CHUNK)], ids_bufs[b], sems[b])
        pltpu.async_copy(seq_hbm.at[pl.ds(st * D, CHUNK * D)], in_bufs[b], sems[b])

    def wait_dmas(b):
        pltpu.make_async_copy(ids_hbm.at[pl.ds(0, CHUNK)], ids_bufs[b], sems[b]).wait()
        pltpu.make_async_copy(seq_hbm.at[pl.ds(0, CHUNK * D)], in_bufs[b], sems[b]).wait()

    def process(j, b):
        ids_v = ids_bufs[b]
        in_v = in_bufs[b]
        st = chunk_start(j)
        lo = jnp.maximum(r0, a0 + j * CHUNK) - st
        hi = jnp.minimum(r1, st + CHUNK) - st

        def pair_body(p, _):
            infos = []
            for gg in range(2):
                g = p * 2 + gg
                idv = ids_v[pl.ds(g * 16, 16)]
                rowi = g * 16 + iota
                inr = (rowi >= lo) & (rowi < hi)
                locv = idv - seg_base
                offv = jnp.where(inr, locv * D, SPW * D)
                cofv = jnp.where(inr, locv * 16, SPW * 16)
                offs = [offv[lane] for lane in range(16)]
                cofs = [cofv[lane] for lane in range(16)]
                infos.append((g, offs, cofs))
            for g, offs, cofs in infos:
                for lane in range(16):
                    rbase = (g * 16 + lane) * D
                    for k in range(D // 16):
                        plsc.addupdate(acc_v.at[pl.ds(offs[lane] + k * 16, 16)],
                                       in_v[pl.ds(rbase + k * 16, 16)])
                    plsc.addupdate(cnt_v.at[pl.ds(cofs[lane], 16)], ones)
            return 0
        lax.fori_loop(lo >> 5, (hi + 31) >> 5, pair_body, 0)

    for b in range(2):
        @pl.when(b < nchunks)
        def _(b=b):
            start_dmas(b, b)

    def pair_body(p, _):
        for b in range(2):
            j = p * 2 + b

            @pl.when(j < nchunks)
            def _(j=j, b=b):
                wait_dmas(b)
                process(j, b)

                @pl.when(j + 2 < nchunks)
                def _(j=j, b=b):
                    start_dmas(j + 2, b)
        return 0
    lax.fori_loop(0, (nchunks + 1) >> 1, pair_body, 0)

    def div_body(s, _):
        cv = cnt_v[pl.ds(s * 16, 16)]
        scale = 1.0 / (cv + 1.0)
        for k in range(D // 16):
            acc_v[pl.ds(s * D + k * 16, 16)] *= scale
        return 0
    lax.fori_loop(0, SPW, div_body, 0)

    pltpu.sync_copy(acc_v.at[pl.ds(0, SPW * D)],
                    out_hbm.at[pl.ds(seg_base * D, SPW * D)])


@jax.jit
def _sc_call(bounds, seqf, ids):
    mesh = plsc.VectorSubcoreMesh(core_axis_name="c", subcore_axis_name="s")
    return pl.kernel(
        _sc_body,
        mesh=mesh,
        out_type=jax.ShapeDtypeStruct((SEG_PAD * D,), jnp.float32),
        scratch_types=[
            pltpu.VMEM((NB,), jnp.int32),
            pltpu.VMEM((CHUNK,), jnp.int32),
            pltpu.VMEM((CHUNK,), jnp.int32),
            pltpu.VMEM((CHUNK * D,), jnp.float32),
            pltpu.VMEM((CHUNK * D,), jnp.float32),
            pltpu.VMEM(((SPW + 1) * D,), jnp.float32),
            pltpu.VMEM(((SPW + 1) * 16,), jnp.float32),
            pltpu.SemaphoreType.DMA,
            pltpu.SemaphoreType.DMA,
        ],
    )(bounds, seqf, ids)


def kernel(seq, sub_match):
    ids = sub_match.astype(jnp.int32)
    marks = jnp.arange(NW + 1, dtype=jnp.int32) * SPW
    bounds = jnp.searchsorted(ids, marks).astype(jnp.int32)
    bounds = jnp.pad(bounds, (0, NB - (NW + 1)))
    out = _sc_call(bounds, seq.reshape(-1), ids)
    return out.reshape(SEG_PAD, D)[:NSEG]


# speedup vs baseline: 8.6599x; 2.3280x over previous
import jax
import jax.numpy as jnp
from jax import lax
from jax.experimental import pallas as pl
from jax.experimental.pallas import tpu as pltpu
from jax.experimental.pallas import tpu_sc as plsc

N_ROWS = 320000
D = 128
NSEG = 10000
NW = 32
SPW = 313
SEG_PAD = NW * SPW
CHUNK = 256
CSHIFT = 8
NB = 48


def _splat_lane(vec, lane):
    idx = jnp.full((16, 1), lane, jnp.int32)
    dn = lax.GatherDimensionNumbers(
        offset_dims=(), collapsed_slice_dims=(0,), start_index_map=(0,))
    return lax.gather(vec, idx, dn, (1,),
                      mode=lax.GatherScatterMode.PROMISE_IN_BOUNDS)


def _sc_body(bounds_hbm, seq_hbm, ids_hbm, out_hbm,
             bounds_v, ids0, ids1, in0, in1, acc_v, cnt_v, sem0, sem1):
    wid = lax.axis_index("s") * 2 + lax.axis_index("c")

    pltpu.sync_copy(bounds_hbm, bounds_v)

    zeros = jnp.zeros((16,), jnp.float32)
    ones = jnp.ones((16,), jnp.float32)
    iota = lax.iota(jnp.int32, 16)

    def zero_body(s, _):
        for k in range(D // 16):
            acc_v[pl.ds(s * D + k * 16, 16)] = zeros
        cnt_v[pl.ds(s * 16, 16)] = zeros
        return 0
    lax.fori_loop(0, SPW, zero_body, 0)

    seg_base = wid * SPW
    bv = bounds_v[pl.ds(wid, 16)]
    r0 = bv[0]
    r1 = bv[1]
    a0 = (r0 >> 3) << 3
    nchunks = (r1 - a0 + (CHUNK - 1)) >> CSHIFT

    ids_bufs = (ids0, ids1)
    in_bufs = (in0, in1)
    sems = (sem0, sem1)

    def chunk_start(j):
        return pl.multiple_of(jnp.minimum(a0 + j * CHUNK, N_ROWS - CHUNK), 8)

    def start_dmas(j, b):
        st = chunk_start(j)
        pltpu.async_copy(ids_hbm.at[pl.ds(st, CHUNK)], ids_bufs[b], sems[b])
        pltpu.async_copy(seq_hbm.at[pl.ds(st * D, CHUNK * D)], in_bufs[b], sems[b])

    def wait_dmas(b):
        pltpu.make_async_copy(ids_hbm.at[pl.ds(0, CHUNK)], ids_bufs[b], sems[b]).wait()
        pltpu.make_async_copy(seq_hbm.at[pl.ds(0, CHUNK * D)], in_bufs[b], sems[b]).wait()

    def process(j, b):
        ids_v = ids_bufs[b]
        in_v = in_bufs[b]
        st = chunk_start(j)
        lo = jnp.maximum(r0, a0 + j * CHUNK) - st
        hi = jnp.minimum(r1, st + CHUNK) - st

        def group_body(g, _):
            idv = ids_v[pl.ds(g * 16, 16)]
            full = (g * 16 >= lo) & (g * 16 + 16 <= hi)
            first = idv[0]
            last = idv[15]
            fsplat = _splat_lane(idv, 0)
            lsplat = _splat_lane(idv, 15)
            mask_a = idv == fsplat
            n_ok = plsc.all_reduce_population_count(mask_a | (idv == lsplat))
            fast = full & (n_ok[0] == 16)

            def fast_path():
                ma_i = mask_a.astype(jnp.int32)
                tot = [jnp.zeros((16,), jnp.float32) for _ in range(D // 16)]
                asum = [jnp.zeros((16,), jnp.float32) for _ in range(D // 16)]
                for lane in range(16):
                    bit = _splat_lane(ma_i, lane) != 0
                    rbase = (g * 16 + lane) * D
                    for k in range(D // 16):
                        row = in_v[pl.ds(rbase + k * 16, 16)]
                        tot[k] = tot[k] + row
                        asum[k] = asum[k] + jnp.where(bit, row, 0.0)
                off_a = (first - seg_base) * D
                off_b = (last - seg_base) * D
                for k in range(D // 16):
                    plsc.addupdate(acc_v.at[pl.ds(off_a + k * 16, 16)], asum[k])
                    plsc.addupdate(acc_v.at[pl.ds(off_b + k * 16, 16)],
                                   tot[k] - asum[k])
                cnt_a = plsc.all_reduce_population_count(mask_a).astype(jnp.float32)
                plsc.addupdate(cnt_v.at[pl.ds((first - seg_base) * 16, 16)], cnt_a)
                plsc.addupdate(cnt_v.at[pl.ds((last - seg_base) * 16, 16)],
                               16.0 - cnt_a)

            def slow_path():
                rowi = g * 16 + iota
                inr = (rowi >= lo) & (rowi < hi)
                locv = idv - seg_base
                offv = jnp.where(inr, locv * D, SPW * D)
                cofv = jnp.where(inr, locv * 16, SPW * 16)
                offs = [offv[lane] for lane in range(16)]
                cofs = [cofv[lane] for lane in range(16)]
                for lane in range(16):
                    rbase = (g * 16 + lane) * D
                    for k in range(D // 16):
                        plsc.addupdate(acc_v.at[pl.ds(offs[lane] + k * 16, 16)],
                                       in_v[pl.ds(rbase + k * 16, 16)])
                    plsc.addupdate(cnt_v.at[pl.ds(cofs[lane], 16)], ones)

            lax.cond(fast, fast_path, slow_path)
            return 0
        lax.fori_loop(lo >> 4, (hi + 15) >> 4, group_body, 0)

    for b in range(2):
        @pl.when(b < nchunks)
        def _(b=b):
            start_dmas(b, b)

    def pair_body(p, _):
        for b in range(2):
            j = p * 2 + b

            @pl.when(j < nchunks)
            def _(j=j, b=b):
                wait_dmas(b)
                process(j, b)

                @pl.when(j + 2 < nchunks)
                def _(j=j, b=b):
                    start_dmas(j + 2, b)
        return 0
    lax.fori_loop(0, (nchunks + 1) >> 1, pair_body, 0)

    def div_body(s, _):
        cv = cnt_v[pl.ds(s * 16, 16)]
        scale = 1.0 / (cv + 1.0)
        for k in range(D // 16):
            acc_v[pl.ds(s * D + k * 16, 16)] *= scale
        return 0
    lax.fori_loop(0, SPW, div_body, 0)

    pltpu.sync_copy(acc_v.at[pl.ds(0, SPW * D)],
                    out_hbm.at[pl.ds(seg_base * D, SPW * D)])


@jax.jit
def _sc_call(bounds, seqf, ids):
    mesh = plsc.VectorSubcoreMesh(core_axis_name="c", subcore_axis_name="s")
    return pl.kernel(
        _sc_body,
        mesh=mesh,
        compiler_params=pltpu.CompilerParams(needs_layout_passes=False),
        out_type=jax.ShapeDtypeStruct((SEG_PAD * D,), jnp.float32),
        scratch_types=[
            pltpu.VMEM((NB,), jnp.int32),
            pltpu.VMEM((CHUNK,), jnp.int32),
            pltpu.VMEM((CHUNK,), jnp.int32),
            pltpu.VMEM((CHUNK * D,), jnp.float32),
            pltpu.VMEM((CHUNK * D,), jnp.float32),
            pltpu.VMEM(((SPW + 1) * D,), jnp.float32),
            pltpu.VMEM(((SPW + 1) * 16,), jnp.float32),
            pltpu.SemaphoreType.DMA,
            pltpu.SemaphoreType.DMA,
        ],
    )(bounds, seqf, ids)


def kernel(seq, sub_match):
    ids = sub_match.astype(jnp.int32)
    marks = jnp.arange(NW + 1, dtype=jnp.int32) * SPW
    bounds = jnp.searchsorted(ids, marks).astype(jnp.int32)
    bounds = jnp.pad(bounds, (0, NB - (NW + 1)))
    out = _sc_call(bounds, seq.reshape(-1), ids)
    return out.reshape(SEG_PAD, D)[:NSEG]
